# baseline (speedup 1.0000x reference)
import jax
import jax.numpy as jnp
from jax import lax
from jax.experimental import pallas as pl
from jax.experimental.pallas import tpu as pltpu
from jax.experimental.pallas import tpu_sc as plsc

NC = 2
NS = 16
NW = NC * NS
L = 16

F = 26
D = 32
G = 4
GW = G * D
CHUNK = 32
M = CHUNK * F
SW = 64
NSTREAM = M // SW

NB = 8
BLK = 128


def _transpose_body(tt_hbm, tg_hbm, *scratch):
    in_bufs = scratch[0:NB]
    out_bufs = scratch[NB:2 * NB]
    tail_buf = scratch[2 * NB]
    in_sems = scratch[2 * NB + 1:3 * NB + 1]
    out_sems = scratch[3 * NB + 1:4 * NB + 1]

    wid = lax.axis_index("s") * NC + lax.axis_index("c")
    v = tt_hbm.shape[1]
    nfull = v // BLK
    per_w = nfull // NW + 1
    b0 = wid * per_w

    row_lo = lax.iota(jnp.int32, L)
    row_hi = row_lo + L

    def in_desc(s, b):
        return pltpu.make_async_copy(
            tt_hbm.at[:, pl.ds(b * BLK, BLK)], in_bufs[s], in_sems[s])

    def out_desc(s, b):
        return pltpu.make_async_copy(
            out_bufs[s], tg_hbm.at[pl.ds(b * (BLK // G), BLK // G)],
            out_sems[s])

    def valid(i):
        return (i >= 0) & (i < per_w) & (b0 + i < nfull)

    for s in range(NB):
        @pl.when(valid(s))
        def _():
            in_desc(s, b0 + s).start()

    def step(i, s):
        b = b0 + i

        @pl.when(valid(i - NB))
        def _():
            out_desc(s, b - NB).wait()

        @pl.when(valid(i))
        def _():
            in_desc(s, b).wait()

            def col_body(c, carry):
                col = jnp.full((L,), c, jnp.int32)
                v0 = plsc.load_gather(in_bufs[s], [row_lo, col])
                v1 = plsc.load_gather(in_bufs[s], [row_hi, col])
                out_bufs[s][c // G, pl.ds((c % G) * D, L)] = v0
                out_bufs[s][c // G, pl.ds((c % G) * D + L, L)] = v1
                return carry
            lax.fori_loop(0, BLK, col_body, 0, unroll=8)

            out_desc(s, b).start()

            @pl.when(valid(i + NB))
            def _():
                in_desc(s, b + NB).start()

    def group_body(gi, carry):
        for s in range(NB):
            step(gi * NB + s, s)
        return carry
    lax.fori_loop(0, per_w // NB, group_body, 0, unroll=False)
    base = (per_w // NB) * NB
    for s in range(per_w % NB):
        step(base + s, s)
    for k in range(NB):
        i = per_w + k

        @pl.when(valid(i - NB))
        def _():
            out_desc(i % NB, b0 + i - NB).wait()

    tail = v - nfull * BLK
    if tail:
        @pl.when(wid == NW - 1)
        def _():
            pltpu.sync_copy(tt_hbm.at[:, pl.ds(nfull * BLK, tail)], tail_buf)

            def tail_body(c, carry):
                col = jnp.full((L,), c, jnp.int32)
                v0 = plsc.load_gather(tail_buf, [row_lo, col])
                v1 = plsc.load_gather(tail_buf, [row_hi, col])
                out_bufs[0][c // G, pl.ds((c % G) * D, L)] = v0
                out_bufs[0][c // G, pl.ds((c % G) * D + L, L)] = v1
                return carry
            lax.fori_loop(0, tail, tail_body, 0, unroll=8)
            pltpu.sync_copy(
                out_bufs[0].at[pl.ds(0, tail // G)],
                tg_hbm.at[pl.ds(nfull * (BLK // G), tail // G)])


def _gather_body(inputs_hbm, table_hbm, offt_hbm, out_hbm,
                 in_v, off_v, idx_v, qoff_v, buf_v, out_v, sem):
    wid = lax.axis_index("s") * NC + lax.axis_index("c")
    n_chunks = out_hbm.shape[0] // (NW * CHUNK)

    pltpu.sync_copy(offt_hbm, off_v)

    def chunk_body(c, carry):
        base = (wid * n_chunks + c) * M
        pltpu.sync_copy(inputs_hbm.at[pl.ds(base, M)], in_v)

        def idx_body(i, carry2):
            s = i * L
            idx = in_v[pl.ds(s, L)] + off_v[pl.ds(s, L)]
            idx_v[pl.ds(s, L)] = idx >> 2
            qoff_v[pl.ds(s, L)] = (idx & 3) * D
            return carry2
        lax.fori_loop(0, M // L, idx_body, 0, unroll=False)

        descs = [
            pltpu.async_copy(table_hbm.at[idx_v.at[pl.ds(j * SW, SW)]],
                             buf_v.at[pl.ds(j * SW, SW)], sem)
            for j in range(NSTREAM)
        ]
        for d in descs:
            d.wait()

        def sum_body(r, carry2):
            g = r * F
            qv0 = qoff_v[pl.ds(g, L)]
            qv1 = qoff_v[pl.ds(g + F - L, L)]
            q0 = qv0[0]
            acc0 = buf_v[g, pl.ds(q0, L)]
            acc1 = buf_v[g, pl.ds(q0 + L, L)]
            for f in range(1, F):
                q = qv0[f] if f < L else qv1[f - (F - L)]
                acc0 = acc0 + buf_v[g + f, pl.ds(q, L)]
                acc1 = acc1 + buf_v[g + f, pl.ds(q + L, L)]
            out_v[r, pl.ds(0, L)] = acc0
            out_v[r, pl.ds(L, L)] = acc1
            return carry2
        lax.fori_loop(0, CHUNK, sum_body, 0, unroll=False)

        pltpu.sync_copy(out_v, out_hbm.at[pl.ds((wid * n_chunks + c) * CHUNK,
                                                CHUNK)])
        return carry

    lax.fori_loop(0, n_chunks, chunk_body, 0, unroll=False)


def kernel(inputs, table, offset):
    B = inputs.shape[0]
    V = table.shape[0]
    inputs_flat = inputs.reshape(B * F)
    off_tiled = jnp.tile(offset, CHUNK)
    tt = table.T

    params = pltpu.CompilerParams(use_tc_tiling_on_sc=True,
                                  needs_layout_passes=False)
    mesh = plsc.VectorSubcoreMesh(core_axis_name="c", subcore_axis_name="s")

    transpose_k = pl.kernel(
        _transpose_body,
        out_type=jax.ShapeDtypeStruct((V // G, GW), jnp.float32),
        mesh=mesh,
        scratch_types=(
            [pltpu.VMEM((D, BLK), jnp.float32)] * NB
            + [pltpu.VMEM((BLK // G, GW), jnp.float32)] * NB
            + [pltpu.VMEM((D, 64), jnp.float32)]
            + [pltpu.SemaphoreType.DMA] * (2 * NB)
        ),
        compiler_params=params,
    )
    table_g = transpose_k(tt)

    gather_k = pl.kernel(
        _gather_body,
        out_type=jax.ShapeDtypeStruct((B, D), jnp.float32),
        mesh=mesh,
        scratch_types=[
            pltpu.VMEM((M,), jnp.int32),
            pltpu.VMEM((M,), jnp.int32),
            pltpu.VMEM((M,), jnp.int32),
            pltpu.VMEM((M,), jnp.int32),
            pltpu.VMEM((M, GW), jnp.float32),
            pltpu.VMEM((CHUNK, D), jnp.float32),
            pltpu.SemaphoreType.DMA,
        ],
        compiler_params=params,
    )
    return gather_k(inputs_flat, table_g, off_tiled)


# speedup vs baseline: 1.4460x; 1.4460x over previous
import jax
import jax.numpy as jnp
from jax import lax
from jax.experimental import pallas as pl
from jax.experimental.pallas import tpu as pltpu
from jax.experimental.pallas import tpu_sc as plsc

NC = 2
NS = 16
NW = NC * NS
L = 16

F = 26
D = 32
G = 4
GW = G * D
CHUNK = 32
M = CHUNK * F
SW = 64
NSTREAM = M // SW


def _body(inputs_hbm, table_hbm, offt_hbm, out_hbm,
          in_v, off_v, idx_v, qoff_v, buf_v, out_v, sem):
    wid = lax.axis_index("s") * NC + lax.axis_index("c")
    n_chunks = out_hbm.shape[0] // (NW * CHUNK)

    pltpu.sync_copy(offt_hbm, off_v)

    def chunk_body(c, carry):
        base = (wid * n_chunks + c) * M
        pltpu.sync_copy(inputs_hbm.at[pl.ds(base, M)], in_v)

        def idx_body(i, carry2):
            s = i * L
            idx = in_v[pl.ds(s, L)] + off_v[pl.ds(s, L)]
            idx_v[pl.ds(s, L)] = idx >> 2
            qoff_v[pl.ds(s, L)] = (idx & 3) * D
            return carry2
        lax.fori_loop(0, M // L, idx_body, 0, unroll=False)

        descs = [
            pltpu.async_copy(table_hbm.at[idx_v.at[pl.ds(j * SW, SW)]],
                             buf_v.at[pl.ds(j * SW, SW)], sem)
            for j in range(NSTREAM)
        ]
        for d in descs:
            d.wait()

        def sum_body(r, carry2):
            g = r * F
            qv0 = qoff_v[pl.ds(g, L)]
            qv1 = qoff_v[pl.ds(g + F - L, L)]
            q0 = qv0[0]
            acc0 = buf_v[g, pl.ds(q0, L)]
            acc1 = buf_v[g, pl.ds(q0 + L, L)]
            for f in range(1, F):
                q = qv0[f] if f < L else qv1[f - (F - L)]
                acc0 = acc0 + buf_v[g + f, pl.ds(q, L)]
                acc1 = acc1 + buf_v[g + f, pl.ds(q + L, L)]
            out_v[r, pl.ds(0, L)] = acc0
            out_v[r, pl.ds(L, L)] = acc1
            return carry2
        lax.fori_loop(0, CHUNK, sum_body, 0, unroll=False)

        pltpu.sync_copy(out_v, out_hbm.at[pl.ds((wid * n_chunks + c) * CHUNK,
                                                CHUNK)])
        return carry

    lax.fori_loop(0, n_chunks, chunk_body, 0, unroll=False)


def kernel(inputs, table, offset):
    B = inputs.shape[0]
    inputs_flat = inputs.reshape(B * F)
    off_tiled = jnp.tile(offset, CHUNK)
    V = table.shape[0]
    table_g = (table.T.reshape(D, V // G, G)
               .transpose(1, 2, 0)
               .reshape(V // G, GW))

    k = pl.kernel(
        _body,
        out_type=jax.ShapeDtypeStruct((B, D), jnp.float32),
        mesh=plsc.VectorSubcoreMesh(core_axis_name="c", subcore_axis_name="s"),
        scratch_types=[
            pltpu.VMEM((M,), jnp.int32),
            pltpu.VMEM((M,), jnp.int32),
            pltpu.VMEM((M,), jnp.int32),
            pltpu.VMEM((M,), jnp.int32),
            pltpu.VMEM((M, GW), jnp.float32),
            pltpu.VMEM((CHUNK, D), jnp.float32),
            pltpu.SemaphoreType.DMA,
        ],
        compiler_params=pltpu.CompilerParams(use_tc_tiling_on_sc=True,
                                             needs_layout_passes=False),
    )
    return k(inputs_flat, table_g, off_tiled)
